# Initial kernel scaffold; baseline (speedup 1.0000x reference)
import jax
import jax.numpy as jnp
from jax.experimental import pallas as pl


def kernel(initial_node_representation, annotations, adj0, adj1, W_hidden, b_hidden, W_msg0, b_msg0, W_msg1, b_msg1, W_ih, W_hh, b_ih, b_hh):
    raise NotImplementedError("write your pallas kernel here")


# speedup vs baseline: 5.7126x; 5.7126x over previous
import functools

import jax
import jax.numpy as jnp
from jax import lax
from jax.experimental import pallas as pl
from jax.experimental.pallas import tpu as pltpu
from jax.experimental.pallas import tpu_sc as plsc

N = 10000
H = 128
A = 16
E = 160000
T = 3

_NC = 2
_NS = 16
_NW = _NC * _NS
_CH = 128
_EPT = (2 * E) // _NW
_CHUNKS = -(-_EPT // _CH)
_EPT_P = _CHUNKS * _CH
_NPAD = 10112
_RPS = _NPAD // _NS
_ACC_ROWS = _NPAD
assert _CHUNKS % 2 == 1 and _CHUNKS >= 3

_BLK = 1000


def _sc_aggregate(ids, table):
    mesh = plsc.VectorSubcoreMesh(core_axis_name="c", subcore_axis_name="s")

    @functools.partial(
        pl.kernel,
        out_type=jax.ShapeDtypeStruct((_NC, _NPAD, H), jnp.float32),
        mesh=mesh,
        scratch_types=[
            pltpu.VMEM((2, 2, _CH), jnp.int32),
            pltpu.VMEM((2, _CH, H), jnp.float32),
            pltpu.VMEM_SHARED((_ACC_ROWS, H), jnp.float32),
            pltpu.SemaphoreType.DMA,
            pltpu.SemaphoreType.DMA,
            pltpu.SemaphoreType.DMA,
            pltpu.SemaphoreType.DMA,
        ],
    )
    def agg(ids_hbm, table_hbm, out_hbm, ibuf, rows, acc,
            isem0, isem1, gsem0, gsem1):
        c = lax.axis_index("c")
        s = lax.axis_index("s")
        wid = c * _NS + s
        isem = (isem0, isem1)
        gsem = (gsem0, gsem1)

        def fire_idx(j, b):
            pltpu.async_copy(ids_hbm.at[wid].at[j], ibuf.at[b], isem[b])

        def wait_idx(b):
            pltpu.make_async_copy(ids_hbm.at[wid].at[0], ibuf.at[b],
                                  isem[b]).wait()

        def fire_gather(b):
            pltpu.async_copy(table_hbm.at[ibuf.at[b].at[0]], rows.at[b],
                             gsem[b])

        def wait_gather(b):
            pltpu.make_async_copy(table_hbm.at[ibuf.at[b].at[0]], rows.at[b],
                                  gsem[b]).wait()

        def scatter(b):
            pltpu.sync_copy(rows.at[b], acc.at[ibuf.at[b].at[1]], add=True)

        def zrow(r, _):
            def zcol(k, _):
                rows[0, r, pl.ds(k * 16, 16)] = jnp.zeros((16,), jnp.float32)
                return 0
            return lax.fori_loop(0, H // 16, zcol, 0)
        lax.fori_loop(0, _CH, zrow, 0)
        base_r = s * _RPS
        nz = _RPS // _CH
        for k in range(nz):
            pltpu.sync_copy(rows.at[0], acc.at[pl.ds(base_r + k * _CH, _CH)])
        rem = _RPS - nz * _CH
        if rem:
            pltpu.sync_copy(rows.at[0].at[pl.ds(0, rem)],
                            acc.at[pl.ds(base_r + nz * _CH, rem)])
        plsc.subcore_barrier()

        pltpu.sync_copy(ids_hbm.at[wid].at[0], ibuf.at[0])
        fire_gather(0)
        fire_idx(1, 1)

        def pair(i, _):
            j0 = 2 * i
            wait_idx(1)
            fire_gather(1)
            wait_gather(0)
            scatter(0)
            fire_idx(j0 + 2, 0)
            wait_idx(0)
            fire_gather(0)
            wait_gather(1)
            scatter(1)
            @pl.when(j0 + 3 < _CHUNKS)
            def _():
                fire_idx(j0 + 3, 1)
            return 0
        lax.fori_loop(0, (_CHUNKS - 1) // 2, pair, 0)
        wait_gather(0)
        scatter(0)

        plsc.subcore_barrier()

        pltpu.sync_copy(acc.at[pl.ds(base_r, _RPS)],
                        out_hbm.at[c].at[pl.ds(base_r, _RPS)])

    return agg(ids, table)


def _init_body(x_ref, ann_ref, wt_ref, b_ref, h_ref):
    h_ref[...] = (
        jnp.dot(x_ref[...], wt_ref[:H], preferred_element_type=jnp.float32)
        + jnp.dot(ann_ref[...], wt_ref[H:], preferred_element_type=jnp.float32)
        + b_ref[...]
    )


def _tc_init(x, ann, wh_t, bh):
    return pl.pallas_call(
        _init_body,
        grid=(N // _BLK,),
        in_specs=[
            pl.BlockSpec((_BLK, H), lambda i: (i, 0)),
            pl.BlockSpec((_BLK, A), lambda i: (i, 0)),
            pl.BlockSpec((H + A, H), lambda i: (0, 0)),
            pl.BlockSpec((1, H), lambda i: (0, 0)),
        ],
        out_specs=pl.BlockSpec((_BLK, H), lambda i: (i, 0)),
        out_shape=jax.ShapeDtypeStruct((N, H), jnp.float32),
    )(x, ann, wh_t, bh)


def _pre_body(h_ref, wcat_ref, bcat_ref, tt_ref, gh_ref):
    out = jnp.dot(h_ref[...], wcat_ref[...], preferred_element_type=jnp.float32)
    out = out + bcat_ref[...]
    tt_ref[0] = out[:, :H]
    tt_ref[1] = out[:, H:2 * H]
    gh_ref[...] = out[:, 2 * H:]


def _tc_pre(h, wcat, bcat):
    return pl.pallas_call(
        _pre_body,
        grid=(N // _BLK,),
        in_specs=[
            pl.BlockSpec((_BLK, H), lambda i: (i, 0)),
            pl.BlockSpec((H, 5 * H), lambda i: (0, 0)),
            pl.BlockSpec((1, 5 * H), lambda i: (0, 0)),
        ],
        out_specs=[
            pl.BlockSpec((2, _BLK, H), lambda i: (0, i, 0)),
            pl.BlockSpec((_BLK, 3 * H), lambda i: (i, 0)),
        ],
        out_shape=[
            jax.ShapeDtypeStruct((2, N, H), jnp.float32),
            jax.ShapeDtypeStruct((N, 3 * H), jnp.float32),
        ],
    )(h, wcat, bcat)


def _post_body(acc_ref, h_ref, gh_ref, wih_t_ref, bih_ref, hnew_ref):
    inc = acc_ref[0] + acc_ref[1]
    gi = jnp.dot(inc, wih_t_ref[...], preferred_element_type=jnp.float32)
    gi = gi + bih_ref[...]
    gh = gh_ref[...]
    r = jax.nn.sigmoid(gi[:, :H] + gh[:, :H])
    z = jax.nn.sigmoid(gi[:, H:2 * H] + gh[:, H:2 * H])
    n = jnp.tanh(gi[:, 2 * H:] + r * gh[:, 2 * H:])
    hnew_ref[...] = (1.0 - z) * n + z * h_ref[...]


def _tc_post(acc, h, gh, wih_t, bih):
    return pl.pallas_call(
        _post_body,
        grid=(N // _BLK,),
        in_specs=[
            pl.BlockSpec((2, _BLK, H), lambda i: (0, i, 0)),
            pl.BlockSpec((_BLK, H), lambda i: (i, 0)),
            pl.BlockSpec((_BLK, 3 * H), lambda i: (i, 0)),
            pl.BlockSpec((H, 3 * H), lambda i: (0, 0)),
            pl.BlockSpec((1, 3 * H), lambda i: (0, 0)),
        ],
        out_specs=pl.BlockSpec((_BLK, H), lambda i: (i, 0)),
        out_shape=jax.ShapeDtypeStruct((N, H), jnp.float32),
    )(acc, h, gh, wih_t, bih)


def kernel(initial_node_representation, annotations, adj0, adj1, W_hidden,
           b_hidden, W_msg0, b_msg0, W_msg1, b_msg1, W_ih, W_hh, b_ih, b_hh):
    wh_t = W_hidden.T
    wcat = jnp.concatenate([W_msg0.T, W_msg1.T, W_hh.T], axis=1)
    bcat = jnp.concatenate([b_msg0, b_msg1, b_hh])[None]
    wih_t = W_ih.T

    src = jnp.concatenate([adj0[:, 0], adj1[:, 0] + N])
    dst = jnp.concatenate([adj0[:, 1], adj1[:, 1]])
    per = (2 * E) // _NW
    src_p = jnp.pad(src.reshape(_NW, per), ((0, 0), (0, _EPT_P - per)),
                    constant_values=0).reshape(_NW, _CHUNKS, _CH)
    dst_p = jnp.pad(dst.reshape(_NW, per), ((0, 0), (0, _EPT_P - per)),
                    constant_values=N).reshape(_NW, _CHUNKS, _CH)
    ids = jnp.stack([src_p, dst_p], axis=2)

    h = _tc_init(initial_node_representation, annotations, wh_t, b_hidden[None])
    for _ in range(T):
        tt, gh = _tc_pre(h, wcat, bcat)
        acc = _sc_aggregate(ids, tt.reshape(2 * N, H))
        h = _tc_post(acc, h, gh, wih_t, b_ih[None])
    return h


# speedup vs baseline: 6.2997x; 1.1028x over previous
import functools

import jax
import jax.numpy as jnp
from jax import lax
from jax.experimental import pallas as pl
from jax.experimental.pallas import tpu as pltpu
from jax.experimental.pallas import tpu_sc as plsc

N = 10000
H = 128
A = 16
E = 160000
T = 3

_NC = 2
_NS = 16
_NW = _NC * _NS
_CH = 112
_EPT = (2 * E) // _NW
_CHUNKS = -(-_EPT // _CH)
_EPT_P = _CHUNKS * _CH
_NPAD = 10112
_RPS = _NPAD // _NS
_ACC_ROWS = _NPAD
assert _CHUNKS % 3 == 0 and _CHUNKS >= 6 and _EPT_P % 8 == 0

_BLK = 1000


def _sc_aggregate(ids, table):
    mesh = plsc.VectorSubcoreMesh(core_axis_name="c", subcore_axis_name="s")

    @functools.partial(
        pl.kernel,
        out_type=jax.ShapeDtypeStruct((_NC, _NPAD, H), jnp.float32),
        mesh=mesh,
        scratch_types=[
            pltpu.VMEM((3, 2, _CH), jnp.int32),
            pltpu.VMEM((3, _CH, H), jnp.float32),
            pltpu.VMEM_SHARED((_ACC_ROWS, H), jnp.float32),
            pltpu.SemaphoreType.DMA,
            pltpu.SemaphoreType.DMA,
            pltpu.SemaphoreType.DMA,
            pltpu.SemaphoreType.DMA,
            pltpu.SemaphoreType.DMA,
            pltpu.SemaphoreType.DMA,
            pltpu.SemaphoreType.DMA,
            pltpu.SemaphoreType.DMA,
            pltpu.SemaphoreType.DMA,
        ],
    )
    def agg(ids_hbm, table_hbm, out_hbm, ibuf, rows, acc,
            isem0, isem1, isem2, gsem0, gsem1, gsem2, ssem0, ssem1, ssem2):
        c = lax.axis_index("c")
        s = lax.axis_index("s")
        wid = c * _NS + s
        isem = (isem0, isem1, isem2)
        gsem = (gsem0, gsem1, gsem2)
        ssem = (ssem0, ssem1, ssem2)

        def fire_idx(j, b):
            pltpu.async_copy(ids_hbm.at[wid].at[j], ibuf.at[b], isem[b])

        def wait_idx(b):
            pltpu.make_async_copy(ids_hbm.at[wid].at[0], ibuf.at[b],
                                  isem[b]).wait()

        def fire_gather(b):
            pltpu.async_copy(table_hbm.at[ibuf.at[b].at[0]], rows.at[b],
                             gsem[b])

        def wait_gather(b):
            pltpu.make_async_copy(table_hbm.at[ibuf.at[b].at[0]], rows.at[b],
                                  gsem[b]).wait()

        def fire_scatter(b):
            pltpu.async_copy(rows.at[b], acc.at[ibuf.at[b].at[1]], ssem[b],
                             add=True)

        def wait_scatter(b):
            pltpu.make_async_copy(rows.at[b], acc.at[ibuf.at[b].at[1]],
                                  ssem[b]).wait()

        def zrow(r, _):
            def zcol(k, _):
                rows[0, r, pl.ds(k * 16, 16)] = jnp.zeros((16,), jnp.float32)
                return 0
            return lax.fori_loop(0, H // 16, zcol, 0)
        lax.fori_loop(0, _CH, zrow, 0)
        base_r = s * _RPS
        nz = _RPS // _CH
        for k in range(nz):
            pltpu.sync_copy(rows.at[0], acc.at[pl.ds(base_r + k * _CH, _CH)])
        rem = _RPS - nz * _CH
        if rem:
            pltpu.sync_copy(rows.at[0].at[pl.ds(0, rem)],
                            acc.at[pl.ds(base_r + nz * _CH, rem)])
        plsc.subcore_barrier()

        pltpu.sync_copy(ids_hbm.at[wid].at[0], ibuf.at[0])
        fire_gather(0)
        fire_idx(1, 1)

        def step(j, b, bp, bn):
            wait_gather(b)
            fire_scatter(b)
            @pl.when(j > 0)
            def _():
                wait_scatter(bp)
            @pl.when(j + 2 < _CHUNKS)
            def _():
                fire_idx(j + 2, bp)
            @pl.when(j + 1 < _CHUNKS)
            def _():
                wait_idx(bn)
                fire_gather(bn)

        def triple(i, _):
            j0 = 3 * i
            step(j0, 0, 2, 1)
            step(j0 + 1, 1, 0, 2)
            step(j0 + 2, 2, 1, 0)
            return 0
        lax.fori_loop(0, _CHUNKS // 3, triple, 0)
        wait_scatter((_CHUNKS - 1) % 3)

        plsc.subcore_barrier()

        pltpu.sync_copy(acc.at[pl.ds(base_r, _RPS)],
                        out_hbm.at[c].at[pl.ds(base_r, _RPS)])

    return agg(ids, table)


def _init_body(x_ref, ann_ref, wt_ref, b_ref, h_ref):
    h_ref[...] = (
        jnp.dot(x_ref[...], wt_ref[:H], preferred_element_type=jnp.float32)
        + jnp.dot(ann_ref[...], wt_ref[H:], preferred_element_type=jnp.float32)
        + b_ref[...]
    )


def _tc_init(x, ann, wh_t, bh):
    return pl.pallas_call(
        _init_body,
        grid=(N // _BLK,),
        in_specs=[
            pl.BlockSpec((_BLK, H), lambda i: (i, 0)),
            pl.BlockSpec((_BLK, A), lambda i: (i, 0)),
            pl.BlockSpec((H + A, H), lambda i: (0, 0)),
            pl.BlockSpec((1, H), lambda i: (0, 0)),
        ],
        out_specs=pl.BlockSpec((_BLK, H), lambda i: (i, 0)),
        out_shape=jax.ShapeDtypeStruct((N, H), jnp.float32),
    )(x, ann, wh_t, bh)


def _pre_body(h_ref, wcat_ref, bcat_ref, tt_ref, gh_ref):
    out = jnp.dot(h_ref[...], wcat_ref[...], preferred_element_type=jnp.float32)
    out = out + bcat_ref[...]
    tt_ref[0] = out[:, :H]
    tt_ref[1] = out[:, H:2 * H]
    gh_ref[...] = out[:, 2 * H:]


def _tc_pre(h, wcat, bcat):
    return pl.pallas_call(
        _pre_body,
        grid=(N // _BLK,),
        in_specs=[
            pl.BlockSpec((_BLK, H), lambda i: (i, 0)),
            pl.BlockSpec((H, 5 * H), lambda i: (0, 0)),
            pl.BlockSpec((1, 5 * H), lambda i: (0, 0)),
        ],
        out_specs=[
            pl.BlockSpec((2, _BLK, H), lambda i: (0, i, 0)),
            pl.BlockSpec((_BLK, 3 * H), lambda i: (i, 0)),
        ],
        out_shape=[
            jax.ShapeDtypeStruct((2, N, H), jnp.float32),
            jax.ShapeDtypeStruct((N, 3 * H), jnp.float32),
        ],
    )(h, wcat, bcat)


def _post_body(acc_ref, h_ref, gh_ref, wih_t_ref, bih_ref, hnew_ref):
    inc = acc_ref[0] + acc_ref[1]
    gi = jnp.dot(inc, wih_t_ref[...], preferred_element_type=jnp.float32)
    gi = gi + bih_ref[...]
    gh = gh_ref[...]
    r = jax.nn.sigmoid(gi[:, :H] + gh[:, :H])
    z = jax.nn.sigmoid(gi[:, H:2 * H] + gh[:, H:2 * H])
    n = jnp.tanh(gi[:, 2 * H:] + r * gh[:, 2 * H:])
    hnew_ref[...] = (1.0 - z) * n + z * h_ref[...]


def _tc_post(acc, h, gh, wih_t, bih):
    return pl.pallas_call(
        _post_body,
        grid=(N // _BLK,),
        in_specs=[
            pl.BlockSpec((2, _BLK, H), lambda i: (0, i, 0)),
            pl.BlockSpec((_BLK, H), lambda i: (i, 0)),
            pl.BlockSpec((_BLK, 3 * H), lambda i: (i, 0)),
            pl.BlockSpec((H, 3 * H), lambda i: (0, 0)),
            pl.BlockSpec((1, 3 * H), lambda i: (0, 0)),
        ],
        out_specs=pl.BlockSpec((_BLK, H), lambda i: (i, 0)),
        out_shape=jax.ShapeDtypeStruct((N, H), jnp.float32),
    )(acc, h, gh, wih_t, bih)


def kernel(initial_node_representation, annotations, adj0, adj1, W_hidden,
           b_hidden, W_msg0, b_msg0, W_msg1, b_msg1, W_ih, W_hh, b_ih, b_hh):
    wh_t = W_hidden.T
    wcat = jnp.concatenate([W_msg0.T, W_msg1.T, W_hh.T], axis=1)
    bcat = jnp.concatenate([b_msg0, b_msg1, b_hh])[None]
    wih_t = W_ih.T

    src = jnp.concatenate([adj0[:, 0], adj1[:, 0] + N])
    dst = jnp.concatenate([adj0[:, 1], adj1[:, 1]])
    per = (2 * E) // _NW
    src_p = jnp.pad(src.reshape(_NW, per), ((0, 0), (0, _EPT_P - per)),
                    constant_values=0).reshape(_NW, _CHUNKS, _CH)
    dst_p = jnp.pad(dst.reshape(_NW, per), ((0, 0), (0, _EPT_P - per)),
                    constant_values=N).reshape(_NW, _CHUNKS, _CH)
    ids = jnp.stack([src_p, dst_p], axis=2)

    h = _tc_init(initial_node_representation, annotations, wh_t, b_hidden[None])
    for _ in range(T):
        tt, gh = _tc_pre(h, wcat, bcat)
        acc = _sc_aggregate(ids, tt.reshape(2 * N, H))
        h = _tc_post(acc, h, gh, wih_t, b_ih[None])
    return h


# speedup vs baseline: 7.1968x; 1.1424x over previous
import functools

import jax
import jax.numpy as jnp
from jax import lax
from jax.experimental import pallas as pl
from jax.experimental.pallas import tpu as pltpu
from jax.experimental.pallas import tpu_sc as plsc

N = 10000
H = 128
A = 16
E = 160000
T = 3

_NC = 2
_NS = 16
_NW = _NC * _NS
_CH = 112
_EPT = (2 * E) // _NW
_CHUNKS = -(-_EPT // _CH)
_EPT_P = _CHUNKS * _CH
_NPAD = 10112
_RPS = _NPAD // _NS
_ACC_ROWS = _NPAD
assert _CHUNKS % 6 == 0 and _CHUNKS >= 12 and _EPT_P % 8 == 0

_BLK = 1000


def _sc_aggregate(ids, table):
    mesh = plsc.VectorSubcoreMesh(core_axis_name="c", subcore_axis_name="s")

    @functools.partial(
        pl.kernel,
        out_type=jax.ShapeDtypeStruct((_NC, _NPAD, H), jnp.float32),
        mesh=mesh,
        scratch_types=[
            pltpu.VMEM((6, 2, _CH), jnp.int32),
            pltpu.VMEM((3, _CH, H), jnp.float32),
            pltpu.VMEM_SHARED((_ACC_ROWS, H), jnp.float32),
        ] + [pltpu.SemaphoreType.DMA] * 12,
    )
    def agg(ids_hbm, table_hbm, out_hbm, ibuf, rows, acc, *sems):
        c = lax.axis_index("c")
        s = lax.axis_index("s")
        wid = c * _NS + s
        isem = sems[0:6]
        gsem = sems[6:9]
        ssem = sems[9:12]

        def fire_idx(j, ib):
            pltpu.async_copy(ids_hbm.at[wid].at[j], ibuf.at[ib], isem[ib])

        def wait_idx(ib):
            pltpu.make_async_copy(ids_hbm.at[wid].at[0], ibuf.at[ib],
                                  isem[ib]).wait()

        def fire_gather(ib, rb):
            pltpu.async_copy(table_hbm.at[ibuf.at[ib].at[0]], rows.at[rb],
                             gsem[rb])

        def wait_gather(ib, rb):
            pltpu.make_async_copy(table_hbm.at[ibuf.at[ib].at[0]],
                                  rows.at[rb], gsem[rb]).wait()

        def fire_scatter(ib, rb):
            pltpu.async_copy(rows.at[rb], acc.at[ibuf.at[ib].at[1]],
                             ssem[rb], add=True)

        def wait_scatter(ib, rb):
            pltpu.make_async_copy(rows.at[rb], acc.at[ibuf.at[ib].at[1]],
                                  ssem[rb]).wait()

        def zrow(r, _):
            def zcol(k, _):
                rows[0, r, pl.ds(k * 16, 16)] = jnp.zeros((16,), jnp.float32)
                return 0
            return lax.fori_loop(0, H // 16, zcol, 0)
        lax.fori_loop(0, _CH, zrow, 0)
        base_r = s * _RPS
        nz = _RPS // _CH
        for k in range(nz):
            pltpu.sync_copy(rows.at[0], acc.at[pl.ds(base_r + k * _CH, _CH)])
        rem = _RPS - nz * _CH
        if rem:
            pltpu.sync_copy(rows.at[0].at[pl.ds(0, rem)],
                            acc.at[pl.ds(base_r + nz * _CH, rem)])
        plsc.subcore_barrier()

        pltpu.sync_copy(ids_hbm.at[wid].at[0], ibuf.at[0])
        for jj in range(1, 5):
            fire_idx(jj, jj)
        fire_gather(0, 0)
        wait_idx(1)
        fire_gather(1, 1)

        def step6(j, jm6, rb):
            rbp = (rb + 2) % 3
            ibp = (jm6 + 5) % 6
            ib2 = (jm6 + 2) % 6
            wait_gather(jm6, rb)
            fire_scatter(jm6, rb)
            @pl.when(j > 0)
            def _():
                wait_scatter(ibp, rbp)
            @pl.when(j + 5 < _CHUNKS)
            def _():
                fire_idx(j + 5, ibp)
            @pl.when(j + 2 < _CHUNKS)
            def _():
                wait_idx(ib2)
                fire_gather(ib2, rbp)

        def six(i, _):
            j0 = 6 * i
            for k in range(6):
                step6(j0 + k, k, k % 3)
            return 0
        lax.fori_loop(0, _CHUNKS // 6, six, 0)
        wait_scatter((_CHUNKS - 1) % 6, (_CHUNKS - 1) % 3)

        plsc.subcore_barrier()

        pltpu.sync_copy(acc.at[pl.ds(base_r, _RPS)],
                        out_hbm.at[c].at[pl.ds(base_r, _RPS)])

    return agg(ids, table)


def _init_body(x_ref, ann_ref, wt_ref, b_ref, h_ref):
    h_ref[...] = (
        jnp.dot(x_ref[...], wt_ref[:H], preferred_element_type=jnp.float32)
        + jnp.dot(ann_ref[...], wt_ref[H:], preferred_element_type=jnp.float32)
        + b_ref[...]
    )


def _tc_init(x, ann, wh_t, bh):
    return pl.pallas_call(
        _init_body,
        grid=(N // _BLK,),
        in_specs=[
            pl.BlockSpec((_BLK, H), lambda i: (i, 0)),
            pl.BlockSpec((_BLK, A), lambda i: (i, 0)),
            pl.BlockSpec((H + A, H), lambda i: (0, 0)),
            pl.BlockSpec((1, H), lambda i: (0, 0)),
        ],
        out_specs=pl.BlockSpec((_BLK, H), lambda i: (i, 0)),
        out_shape=jax.ShapeDtypeStruct((N, H), jnp.float32),
    )(x, ann, wh_t, bh)


def _pre_body(h_ref, wcat_ref, bcat_ref, tt_ref, gh_ref):
    out = jnp.dot(h_ref[...], wcat_ref[...], preferred_element_type=jnp.float32)
    out = out + bcat_ref[...]
    tt_ref[0] = out[:, :H]
    tt_ref[1] = out[:, H:2 * H]
    gh_ref[...] = out[:, 2 * H:]


def _tc_pre(h, wcat, bcat):
    return pl.pallas_call(
        _pre_body,
        grid=(N // _BLK,),
        in_specs=[
            pl.BlockSpec((_BLK, H), lambda i: (i, 0)),
            pl.BlockSpec((H, 5 * H), lambda i: (0, 0)),
            pl.BlockSpec((1, 5 * H), lambda i: (0, 0)),
        ],
        out_specs=[
            pl.BlockSpec((2, _BLK, H), lambda i: (0, i, 0)),
            pl.BlockSpec((_BLK, 3 * H), lambda i: (i, 0)),
        ],
        out_shape=[
            jax.ShapeDtypeStruct((2, N, H), jnp.float32),
            jax.ShapeDtypeStruct((N, 3 * H), jnp.float32),
        ],
    )(h, wcat, bcat)


def _post_body(acc_ref, h_ref, gh_ref, wih_t_ref, bih_ref, hnew_ref):
    inc = acc_ref[0] + acc_ref[1]
    gi = jnp.dot(inc, wih_t_ref[...], preferred_element_type=jnp.float32)
    gi = gi + bih_ref[...]
    gh = gh_ref[...]
    r = jax.nn.sigmoid(gi[:, :H] + gh[:, :H])
    z = jax.nn.sigmoid(gi[:, H:2 * H] + gh[:, H:2 * H])
    n = jnp.tanh(gi[:, 2 * H:] + r * gh[:, 2 * H:])
    hnew_ref[...] = (1.0 - z) * n + z * h_ref[...]


def _tc_post(acc, h, gh, wih_t, bih):
    return pl.pallas_call(
        _post_body,
        grid=(N // _BLK,),
        in_specs=[
            pl.BlockSpec((2, _BLK, H), lambda i: (0, i, 0)),
            pl.BlockSpec((_BLK, H), lambda i: (i, 0)),
            pl.BlockSpec((_BLK, 3 * H), lambda i: (i, 0)),
            pl.BlockSpec((H, 3 * H), lambda i: (0, 0)),
            pl.BlockSpec((1, 3 * H), lambda i: (0, 0)),
        ],
        out_specs=pl.BlockSpec((_BLK, H), lambda i: (i, 0)),
        out_shape=jax.ShapeDtypeStruct((N, H), jnp.float32),
    )(acc, h, gh, wih_t, bih)


def kernel(initial_node_representation, annotations, adj0, adj1, W_hidden,
           b_hidden, W_msg0, b_msg0, W_msg1, b_msg1, W_ih, W_hh, b_ih, b_hh):
    wh_t = W_hidden.T
    wcat = jnp.concatenate([W_msg0.T, W_msg1.T, W_hh.T], axis=1)
    bcat = jnp.concatenate([b_msg0, b_msg1, b_hh])[None]
    wih_t = W_ih.T

    src = jnp.concatenate([adj0[:, 0], adj1[:, 0] + N])
    dst = jnp.concatenate([adj0[:, 1], adj1[:, 1]])
    per = (2 * E) // _NW
    src_p = jnp.pad(src.reshape(_NW, per), ((0, 0), (0, _EPT_P - per)),
                    constant_values=0).reshape(_NW, _CHUNKS, _CH)
    dst_p = jnp.pad(dst.reshape(_NW, per), ((0, 0), (0, _EPT_P - per)),
                    constant_values=N).reshape(_NW, _CHUNKS, _CH)
    ids = jnp.stack([src_p, dst_p], axis=2)

    h = _tc_init(initial_node_representation, annotations, wh_t, b_hidden[None])
    for _ in range(T):
        tt, gh = _tc_pre(h, wcat, bcat)
        acc = _sc_aggregate(ids, tt.reshape(2 * N, H))
        h = _tc_post(acc, h, gh, wih_t, b_ih[None])
    return h
